# baseline (speedup 1.0000x reference)
import functools

import jax
import jax.numpy as jnp
from jax import lax
from jax.experimental import pallas as pl
from jax.experimental.pallas import tpu as pltpu
from jax.experimental.pallas import tpu_sc as plsc

N = 512
NAME_DIM = 64
TYPE_DIM = 16
IN_DIM = 112
HID = 128
NCLS = 8

_SC_NC = 1
_SC_NS = 16
_SC_NW = _SC_NC * _SC_NS
_ROWS_PER_W = N // _SC_NW


def _sc_gather_body(name_emb, name_idx, name_out, nidx_v, nrows_v, sem):
    wid = lax.axis_index("s") * _SC_NC + lax.axis_index("c")
    base = wid * _ROWS_PER_W
    pltpu.sync_copy(name_idx.at[pl.ds(base, _ROWS_PER_W)], nidx_v)
    pltpu.async_copy(name_emb.at[nidx_v], nrows_v, sem).wait()
    pltpu.sync_copy(nrows_v, name_out.at[pl.ds(base, _ROWS_PER_W)])


@functools.cache
def _sc_gather():
    return pl.kernel(
        _sc_gather_body,
        out_type=jax.ShapeDtypeStruct((N, NAME_DIM), jnp.float32),
        mesh=plsc.VectorSubcoreMesh(
            core_axis_name="c", subcore_axis_name="s", num_cores=_SC_NC),
        scratch_types=[
            pltpu.VMEM((_ROWS_PER_W,), jnp.int32),
            pltpu.VMEM((_ROWS_PER_W, NAME_DIM), jnp.float32),
            pltpu.SemaphoreType.DMA,
        ],
        compiler_params=pltpu.CompilerParams(use_tc_tiling_on_sc=False),
    )


def _tc_body(adj_ref, nrows_ref, tidx_ref, temb_ref, beh_ref,
             w0_ref, b0_ref, w1_ref, b1_ref,
             wout_ref, bout_ref, pred_ref):
    adj = adj_ref[...]
    w = jnp.where(adj > 0.0, adj, 0.0)
    deg = jnp.maximum(jnp.sum(w, axis=0, keepdims=True), 1e-6)
    deg_col = deg.T

    dn = (((0,), (0,)), ((), ()))

    oh = (tidx_ref[...] ==
          lax.broadcasted_iota(jnp.int32, (N, TYPE_DIM), 1)).astype(jnp.float32)
    trows = jnp.dot(oh, temb_ref[...], preferred_element_type=jnp.float32)
    x = jnp.concatenate([nrows_ref[...], trows, beh_ref[...]], axis=-1)

    t1 = lax.dot_general(w, x, dn, preferred_element_type=jnp.float32)
    h1 = jnp.maximum(
        jnp.dot(t1 / deg_col, w0_ref[...],
                preferred_element_type=jnp.float32) + b0_ref[...], 0.0)
    t2 = lax.dot_general(w, h1, dn,
                         preferred_element_type=jnp.float32)
    h2 = jnp.maximum(
        jnp.dot(t2 / deg_col, w1_ref[...],
                preferred_element_type=jnp.float32) + b1_ref[...], 0.0)
    g = jnp.mean(h2, axis=0, keepdims=True)
    pred_ref[...] = jnp.dot(g, wout_ref[...],
                            preferred_element_type=jnp.float32) + bout_ref[...]


def kernel(x_tensor, adj_tensor, name_idx, type_idx, behavior_feats,
           name_emb, type_emb, W0, b0, W1, b1, Wout, bout):
    adj = adj_tensor.reshape(N, N)

    name_rows = _sc_gather()(name_emb, name_idx)

    pred = pl.pallas_call(
        _tc_body,
        out_shape=jax.ShapeDtypeStruct((1, NCLS), jnp.float32),
    )(adj, name_rows, type_idx.reshape(N, 1), type_emb, behavior_feats,
      W0, b0.reshape(1, HID), W1, b1.reshape(1, HID),
      Wout, bout.reshape(1, NCLS))

    return (pred, adj)


# speedup vs baseline: 1.0394x; 1.0394x over previous
import functools

import jax
import jax.numpy as jnp
from jax import lax
from jax.experimental import pallas as pl
from jax.experimental.pallas import tpu as pltpu
from jax.experimental.pallas import tpu_sc as plsc

N = 512
NAME_DIM = 64
TYPE_DIM = 16
IN_DIM = 112
HID = 128
NCLS = 8

_SC_NC = 1
_SC_NS = 16
_SC_NW = _SC_NC * _SC_NS
_ROWS_PER_W = N // _SC_NW


def _sc_gather_body(name_emb, name_idx, name_out, nidx_v, nrows_v, sem):
    wid = lax.axis_index("s") * _SC_NC + lax.axis_index("c")
    base = wid * _ROWS_PER_W
    pltpu.sync_copy(name_idx.at[pl.ds(base, _ROWS_PER_W)], nidx_v)
    pltpu.async_copy(name_emb.at[nidx_v], nrows_v, sem).wait()
    pltpu.sync_copy(nrows_v, name_out.at[pl.ds(base, _ROWS_PER_W)])


@functools.cache
def _sc_gather():
    return pl.kernel(
        _sc_gather_body,
        out_type=jax.ShapeDtypeStruct((N, NAME_DIM), jnp.float32),
        mesh=plsc.VectorSubcoreMesh(
            core_axis_name="c", subcore_axis_name="s", num_cores=_SC_NC),
        scratch_types=[
            pltpu.VMEM((_ROWS_PER_W,), jnp.int32),
            pltpu.VMEM((_ROWS_PER_W, NAME_DIM), jnp.float32),
            pltpu.SemaphoreType.DMA,
        ],
        compiler_params=pltpu.CompilerParams(use_tc_tiling_on_sc=False),
    )


def _tc_body(adj_ref, nrows_ref, tidx_ref, temb_ref, beh_ref,
             w0_ref, b0_ref, w1_ref, b1_ref,
             wout_ref, bout_ref, pred_ref, adj_out_ref):
    adj = adj_ref[...]
    adj_out_ref[...] = adj
    w = jnp.where(adj > 0.0, adj, 0.0)
    deg = jnp.maximum(jnp.sum(w, axis=0, keepdims=True), 1e-6)
    deg_col = deg.T

    dn = (((0,), (0,)), ((), ()))

    oh = (tidx_ref[...] ==
          lax.broadcasted_iota(jnp.int32, (N, TYPE_DIM), 1)).astype(jnp.float32)
    trows = jnp.dot(oh, temb_ref[...], preferred_element_type=jnp.float32)
    x = jnp.concatenate([nrows_ref[...], trows, beh_ref[...]], axis=-1)

    t1 = lax.dot_general(w, x, dn, preferred_element_type=jnp.float32)
    h1 = jnp.maximum(
        jnp.dot(t1 / deg_col, w0_ref[...],
                preferred_element_type=jnp.float32) + b0_ref[...], 0.0)
    t2 = lax.dot_general(w, h1, dn,
                         preferred_element_type=jnp.float32)
    h2 = jnp.maximum(
        jnp.dot(t2 / deg_col, w1_ref[...],
                preferred_element_type=jnp.float32) + b1_ref[...], 0.0)
    g = jnp.mean(h2, axis=0, keepdims=True)
    pred_ref[...] = jnp.dot(g, wout_ref[...],
                            preferred_element_type=jnp.float32) + bout_ref[...]


def kernel(x_tensor, adj_tensor, name_idx, type_idx, behavior_feats,
           name_emb, type_emb, W0, b0, W1, b1, Wout, bout):
    adj = adj_tensor.reshape(N, N)

    name_rows = _sc_gather()(name_emb, name_idx)

    pred, adj_out = pl.pallas_call(
        _tc_body,
        out_shape=(
            jax.ShapeDtypeStruct((1, NCLS), jnp.float32),
            jax.ShapeDtypeStruct((N, N), jnp.float32),
        ),
    )(adj, name_rows, type_idx.reshape(N, 1), type_emb, behavior_feats,
      W0, b0.reshape(1, HID), W1, b1.reshape(1, HID),
      Wout, bout.reshape(1, NCLS))

    return (pred, adj_out)
